# baseline (speedup 1.0000x reference)
import jax
import jax.numpy as jnp
from jax.experimental import pallas as pl
from jax.experimental.pallas import tpu as pltpu

N = 1024
HID = 128
V = 2
F = HID // 2
SPLITS = 4

_NORMAL = (((1,), (0,)), ((), ()))
_T_DIMNUMS = (((0,), (0,)), ((), ()))


def _bignn_kernel(x_ref, afw_ref, abw_ref, wfw_ref, bfw_ref, wbw_ref,
                  bbw_ref, w1_ref, b1_ref, out_ref,
                  hfw_ref, hbw_ref, buf_fw, buf_bw, sem_fw, sem_bw):
    def _copies():
        H = N // SPLITS
        cs = []
        for src, buf, sem in ((afw_ref, buf_fw, sem_fw),
                              (abw_ref, buf_bw, sem_bw)):
            for i in range(V):
                for s in range(SPLITS):
                    r = pl.ds(s * H, H)
                    cs.append(pltpu.make_async_copy(
                        src.at[i, r, :], buf.at[i, r, :], sem.at[i, s]))
        return cs

    for c in _copies():
        c.start()

    x = x_ref[...]
    for w_ref, b_ref, h_ref in ((wfw_ref, bfw_ref, hfw_ref),
                                (wbw_ref, bbw_ref, hbw_ref)):
        for i in range(V):
            h = (jnp.dot(x, w_ref[i], preferred_element_type=jnp.float32)
                 + b_ref[i:i + 1, :])
            h_ref[:, pl.ds(i * N, N)] = jnp.swapaxes(h, 0, 1)

    for c in _copies():
        c.wait()

    parts = []
    for buf, h_ref in ((buf_bw, hbw_ref), (buf_fw, hfw_ref)):
        acc = None
        for i in range(V):
            a = buf[i].astype(jnp.float32)
            agg_t = jax.lax.dot_general(
                h_ref[:, pl.ds(i * N, N)], a, _NORMAL,
                precision=jax.lax.Precision.DEFAULT,
                preferred_element_type=jnp.float32)
            r = jnp.maximum(agg_t, 0.0)
            acc = r if acc is None else acc + r
        parts.append(acc)
    summed_t = jnp.concatenate(parts, axis=0)

    out_ref[...] = (
        jax.lax.dot_general(summed_t, w1_ref[...], _T_DIMNUMS,
                            preferred_element_type=jnp.float32)
        + b1_ref[...] + x)


@jax.jit
def kernel(inps, fw_adjs, bw_adjs, W_fw, b_fw, W_bw, b_bw, W1, b1):
    out = pl.pallas_call(
        _bignn_kernel,
        in_specs=[
            pl.BlockSpec(memory_space=pltpu.MemorySpace.VMEM),
            pl.BlockSpec(memory_space=pl.ANY),
            pl.BlockSpec(memory_space=pl.ANY),
            pl.BlockSpec(memory_space=pltpu.MemorySpace.VMEM),
            pl.BlockSpec(memory_space=pltpu.MemorySpace.VMEM),
            pl.BlockSpec(memory_space=pltpu.MemorySpace.VMEM),
            pl.BlockSpec(memory_space=pltpu.MemorySpace.VMEM),
            pl.BlockSpec(memory_space=pltpu.MemorySpace.VMEM),
            pl.BlockSpec(memory_space=pltpu.MemorySpace.VMEM),
        ],
        out_specs=pl.BlockSpec(memory_space=pltpu.MemorySpace.VMEM),
        out_shape=jax.ShapeDtypeStruct((N, HID), jnp.float32),
        scratch_shapes=[
            pltpu.VMEM((F, V * N), jnp.float32),
            pltpu.VMEM((F, V * N), jnp.float32),
            pltpu.VMEM((V, N, N), jnp.int32),
            pltpu.VMEM((V, N, N), jnp.int32),
            pltpu.SemaphoreType.DMA((V, SPLITS)),
            pltpu.SemaphoreType.DMA((V, SPLITS)),
        ],
    )(inps, fw_adjs, bw_adjs, W_fw, b_fw, W_bw, b_bw, W1,
      b1.reshape(1, HID))
    return out


# speedup vs baseline: 1.0952x; 1.0952x over previous
import jax
import jax.numpy as jnp
from jax.experimental import pallas as pl
from jax.experimental.pallas import tpu as pltpu

N = 1024
HID = 128
V = 2
F = HID // 2
BLOCK_D = 256
GRID = N // BLOCK_D
NBUF = 3
SPLITS = 2

_NORMAL = (((1,), (0,)), ((), ()))
_T_DIMNUMS = (((0,), (0,)), ((), ()))


def _bignn_kernel(x_ref, afw_ref, abw_ref, wfw_ref, bfw_ref, wbw_ref,
                  bbw_ref, w1_ref, b1_ref, out_ref,
                  hfw_ref, hbw_ref, buf_fw, buf_bw, sem_fw, sem_bw):
    x = x_ref[...]
    for w_ref, b_ref, h_ref in ((wfw_ref, bfw_ref, hfw_ref),
                                (wbw_ref, bbw_ref, hbw_ref)):
        for i in range(V):
            h = (jnp.dot(x, w_ref[i], preferred_element_type=jnp.float32)
                 + b_ref[i:i + 1, :])
            h_ref[:, pl.ds(i * N, N)] = jnp.swapaxes(
                h.astype(jnp.bfloat16), 0, 1)

    def _copies(j, slot):
        d = pl.ds(j * BLOCK_D, BLOCK_D)
        H = N // SPLITS
        cs = []
        for src, buf, sem in ((afw_ref, buf_fw, sem_fw),
                              (abw_ref, buf_bw, sem_bw)):
            for i in range(V):
                for s in range(SPLITS):
                    r = pl.ds(s * H, H)
                    cs.append(pltpu.make_async_copy(
                        src.at[i, r, d], buf.at[slot, i, r],
                        sem.at[slot, i, s]))
        return cs

    def _copy(j, slot):
        for c in _copies(j, slot):
            c.start()

    for j in range(NBUF - 1):
        _copy(j, j)
    for j in range(GRID):
        slot = j % NBUF
        if j + NBUF - 1 < GRID:
            _copy(j + NBUF - 1, (j + NBUF - 1) % NBUF)
        for c in _copies(j, slot):
            c.wait()

        parts = []
        for buf, h_ref in ((buf_bw, hbw_ref), (buf_fw, hfw_ref)):
            acc = None
            for i in range(V):
                a = buf[slot, i].astype(jnp.bfloat16)
                agg_t = jax.lax.dot_general(
                    h_ref[:, pl.ds(i * N, N)], a, _NORMAL,
                    preferred_element_type=jnp.float32)
                r = jnp.maximum(agg_t, 0.0)
                acc = r if acc is None else acc + r
            parts.append(acc)
        summed_t = jnp.concatenate(parts, axis=0)

        d = pl.ds(j * BLOCK_D, BLOCK_D)
        feats = (jax.lax.dot_general(summed_t, w1_ref[...], _T_DIMNUMS,
                                     preferred_element_type=jnp.float32)
                 + b1_ref[...] + x_ref[d, :])
        out_ref[d, :] = feats


@jax.jit
def kernel(inps, fw_adjs, bw_adjs, W_fw, b_fw, W_bw, b_bw, W1, b1):
    out = pl.pallas_call(
        _bignn_kernel,
        in_specs=[
            pl.BlockSpec(memory_space=pltpu.MemorySpace.VMEM),
            pl.BlockSpec(memory_space=pl.ANY),
            pl.BlockSpec(memory_space=pl.ANY),
            pl.BlockSpec(memory_space=pltpu.MemorySpace.VMEM),
            pl.BlockSpec(memory_space=pltpu.MemorySpace.VMEM),
            pl.BlockSpec(memory_space=pltpu.MemorySpace.VMEM),
            pl.BlockSpec(memory_space=pltpu.MemorySpace.VMEM),
            pl.BlockSpec(memory_space=pltpu.MemorySpace.VMEM),
            pl.BlockSpec(memory_space=pltpu.MemorySpace.VMEM),
        ],
        out_specs=pl.BlockSpec(memory_space=pltpu.MemorySpace.VMEM),
        out_shape=jax.ShapeDtypeStruct((N, HID), jnp.float32),
        scratch_shapes=[
            pltpu.VMEM((F, V * N), jnp.bfloat16),
            pltpu.VMEM((F, V * N), jnp.bfloat16),
            pltpu.VMEM((NBUF, V, N, BLOCK_D), jnp.int32),
            pltpu.VMEM((NBUF, V, N, BLOCK_D), jnp.int32),
            pltpu.SemaphoreType.DMA((NBUF, V, SPLITS)),
            pltpu.SemaphoreType.DMA((NBUF, V, SPLITS)),
        ],
    )(inps, fw_adjs, bw_adjs, W_fw, b_fw, W_bw, b_bw, W1,
      b1.reshape(1, HID))
    return out
